# baseline (speedup 1.0000x reference)
import functools

import jax
import jax.numpy as jnp
from jax import lax
from jax.experimental import pallas as pl
from jax.experimental.pallas import tpu as pltpu
from jax.experimental.pallas import tpu_sc as plsc

_D = 256
_N = 8192
_B = 8192
_BM = 1024
_BN = 1024
_GM = _B // _BM
_GN = _N // _BN
_LOSS_SCALE = 1.25 / (_B * _D)

_NW = 32
_RPW = _B // _NW
_CH = 128
_NCH = _RPW // _CH


def _merge(v1, i1, v2, i2):
    take2 = jnp.logical_or(v2 < v1, jnp.logical_and(v2 == v1, i2 < i1))
    return jnp.where(take2, v2, v1), jnp.where(take2, i2, i1)


def _consume(mm_ref, c, s_z_bc, w2_ref, minv_ref, idx_ref):
    sub_iota = lax.broadcasted_iota(jnp.int32, (8, _BM), 0)
    inf = jnp.full((8, _BM), jnp.inf, jnp.float32)
    zero = jnp.zeros((8, _BM), jnp.int32)
    accs = [[inf, zero], [inf, zero]]
    for j in range(_BN // 8):
        mm_j = mm_ref[pl.ds(8 * j, 8), :]
        w2_j = w2_ref[pl.ds(c * _BN + 8 * j, 8), :]
        v = (s_z_bc - mm_j) + jnp.broadcast_to(w2_j, (8, _BM))
        a = accs[j % 2]
        lt = v < a[0]
        a[0] = jnp.minimum(a[0], v)
        a[1] = jnp.where(lt, sub_iota + (8 * j), a[1])

    acc_val, acc_idx = _merge(accs[0][0], accs[0][1], accs[1][0], accs[1][1])
    for h in (4, 2, 1):
        acc_val, acc_idx = _merge(acc_val[:h], acc_idx[:h],
                                  acc_val[h:2 * h], acc_idx[h:2 * h])

    cmin = acc_val
    cidx = acc_idx + c * _BN

    better = jnp.logical_or(c == 0, cmin < minv_ref[...])
    new_min = jnp.where(better, cmin, minv_ref[...])
    minv_ref[...] = new_min
    idx_ref[...] = jnp.where(better[None], cidx[None], idx_ref[...])


_TOT = (_B // _BM) * (_N // _BN) - 1


def _dist_argmin_body(z_ref, w_ref, idx_ref, loss_ref,
                      minv_ref, w2_ref, mm0_ref, mm1_ref):
    m = pl.program_id(0)

    @pl.when(m == 0)
    def _():
        for c in range(_GN):
            w_blk0 = w_ref[pl.ds(c * _BN, _BN), :]
            w2_ref[pl.ds(c * _BN, _BN), :] = jnp.sum(
                w_blk0 * w_blk0, axis=1, keepdims=True)

    z_blk = z_ref[pl.ds(m * _BM, _BM), :]
    zsq = z_blk * z_blk
    ones_row = jnp.ones((1, _D), jnp.float32)
    s_z = lax.dot_general(ones_row, zsq, (((1,), (1,)), ((), ())),
                          preferred_element_type=jnp.float32)
    s_z_bc = jnp.broadcast_to(s_z, (8, _BM))

    def _dot(h, out_ref):
        h = jnp.minimum(h, _TOT)
        zb = z_ref[pl.ds((h // _GN) * _BM, _BM), :]
        w_blk = w_ref[pl.ds((h % _GN) * _BN, _BN), :]
        out_ref[...] = lax.dot_general(
            w_blk, zb + zb, (((1,), (1,)), ((), ())),
            preferred_element_type=jnp.float32)

    g0 = m * _GN

    @pl.when(m == 0)
    def _():
        _dot(0, mm0_ref)

    def _pipe(k, carry):
        c0 = 2 * k
        _dot(g0 + c0 + 1, mm1_ref)
        _consume(mm0_ref, c0, s_z_bc, w2_ref, minv_ref, idx_ref)
        _dot(g0 + c0 + 2, mm0_ref)
        _consume(mm1_ref, c0 + 1, s_z_bc, w2_ref, minv_ref, idx_ref)
        return carry

    lax.fori_loop(0, _GN // 2, _pipe, 0, unroll=2)

    part = jnp.sum(minv_ref[...])
    acc = jnp.where(m == 0, 0.0, loss_ref[0, 0]) + part
    loss_ref[0, 0] = jnp.where(m == _GM - 1, acc * _LOSS_SCALE, acc)


_dist_argmin = pl.pallas_call(
    _dist_argmin_body,
    grid=(_GM,),
    in_specs=[
        pl.BlockSpec((_B, _D), lambda m: (0, 0)),
        pl.BlockSpec((_N, _D), lambda m: (0, 0)),
    ],
    out_specs=[
        pl.BlockSpec((1, 1, _BM), lambda m: (m, 0, 0)),
        pl.BlockSpec(memory_space=pltpu.SMEM),
    ],
    out_shape=[
        jax.ShapeDtypeStruct((_GM, 1, _BM), jnp.int32),
        jax.ShapeDtypeStruct((1, 1), jnp.float32),
    ],
    scratch_shapes=[
        pltpu.VMEM((1, _BM), jnp.float32),
        pltpu.VMEM((_N, 1), jnp.float32),
        pltpu.VMEM((_BN, _BM), jnp.float32),
        pltpu.VMEM((_BN, _BM), jnp.float32),
    ],
    compiler_params=pltpu.CompilerParams(
        dimension_semantics=("arbitrary",),
    ),
)


@functools.cache
def _sc_gather_fn():
    @functools.partial(
        pl.kernel,
        mesh=plsc.VectorSubcoreMesh(core_axis_name="c", subcore_axis_name="s"),
        out_type=jax.ShapeDtypeStruct((_B, _D), jnp.float32),
        scratch_types=[
            pltpu.VMEM((_NCH, _CH), jnp.int32),
            pltpu.VMEM((_RPW, _D), jnp.float32),
            pltpu.SemaphoreType.DMA,
        ],
    )
    def _sc_gather(w_hbm, idx_hbm, out_hbm, idx_v, rows_v, sem):
        wid = lax.axis_index("s") * 2 + lax.axis_index("c")
        pltpu.sync_copy(idx_hbm.at[pl.ds(wid * _NCH, _NCH)], idx_v)
        for k in range(_NCH):
            pltpu.async_copy(w_hbm.at[idx_v.at[k]],
                             rows_v.at[pl.ds(k * _CH, _CH)], sem).wait()
        pltpu.sync_copy(rows_v, out_hbm.at[pl.ds(wid * _RPW, _RPW)])

    return _sc_gather


def kernel(z, W):
    z_flat = z.reshape(-1, _D)
    idx, loss = _dist_argmin(z_flat, W)
    z_q = _sc_gather_fn()(W, idx.reshape(_B // _CH, _CH))
    return (z_q.reshape(z.shape), loss[0, 0])


# speedup vs baseline: 1.0379x; 1.0379x over previous
import functools

import jax
import jax.numpy as jnp
from jax import lax
from jax.experimental import pallas as pl
from jax.experimental.pallas import tpu as pltpu
from jax.experimental.pallas import tpu_sc as plsc

_D = 256
_N = 8192
_B = 8192
_BM = 1024
_BN = 1024
_GM = _B // _BM
_GN = _N // _BN
_LOSS_SCALE = 1.25 / (_B * _D)

_NW = 32
_RPW = _B // _NW
_CH = 128
_NCH = _RPW // _CH


def _merge(v1, i1, v2, i2):
    take2 = jnp.logical_or(v2 < v1, jnp.logical_and(v2 == v1, i2 < i1))
    return jnp.where(take2, v2, v1), jnp.where(take2, i2, i1)


def _consume(mm_ref, c, s_z_bc, w2_ref, minv_ref, idx_ref):
    sub_iota = lax.broadcasted_iota(jnp.int32, (8, _BM), 0)
    inf = jnp.full((8, _BM), jnp.inf, jnp.float32)
    zero = jnp.zeros((8, _BM), jnp.int32)
    accs = [[inf, zero], [inf, zero]]
    for j in range(_BN // 8):
        mm_j = mm_ref[pl.ds(8 * j, 8), :]
        w2_j = w2_ref[pl.ds(c * _BN + 8 * j, 8), :]
        v = (s_z_bc - mm_j) + jnp.broadcast_to(w2_j, (8, _BM))
        a = accs[j % 2]
        lt = v < a[0]
        a[0] = jnp.minimum(a[0], v)
        a[1] = jnp.where(lt, sub_iota + (8 * j), a[1])

    acc_val, acc_idx = _merge(accs[0][0], accs[0][1], accs[1][0], accs[1][1])
    for h in (4, 2, 1):
        acc_val, acc_idx = _merge(acc_val[:h], acc_idx[:h],
                                  acc_val[h:2 * h], acc_idx[h:2 * h])

    cmin = acc_val
    cidx = acc_idx + c * _BN

    better = jnp.logical_or(c == 0, cmin < minv_ref[...])
    new_min = jnp.where(better, cmin, minv_ref[...])
    minv_ref[...] = new_min
    idx_ref[...] = jnp.where(better[None], cidx[None], idx_ref[...])


def _dist_argmin_body(z_ref, w_ref, idx_ref, loss_ref,
                      minv_ref, w2_ref, mm0_ref, mm1_ref):
    m = pl.program_id(0)

    @pl.when(m == 0)
    def _():
        for c in range(_GN):
            w_blk0 = w_ref[pl.ds(c * _BN, _BN), :]
            w2_ref[pl.ds(c * _BN, _BN), :] = jnp.sum(
                w_blk0 * w_blk0, axis=1, keepdims=True)

    zsq = z_ref[...] * z_ref[...]
    ones_row = jnp.ones((1, _D), jnp.float32)
    s_z = lax.dot_general(ones_row, zsq, (((1,), (1,)), ((), ())),
                          preferred_element_type=jnp.float32)
    s_z_bc = jnp.broadcast_to(s_z, (8, _BM))

    z2_blk = z_ref[...] + z_ref[...]

    def _dot(c, out_ref):
        w_blk = w_ref[pl.ds(c * _BN, _BN), :]
        out_ref[...] = lax.dot_general(
            w_blk, z2_blk, (((1,), (1,)), ((), ())),
            preferred_element_type=jnp.float32)

    _dot(0, mm0_ref)

    def _pipe(k, carry):
        c0 = 2 * k
        _dot(c0 + 1, mm1_ref)
        _consume(mm0_ref, c0, s_z_bc, w2_ref, minv_ref, idx_ref)
        _dot(jnp.minimum(c0 + 2, _GN - 1), mm0_ref)
        _consume(mm1_ref, c0 + 1, s_z_bc, w2_ref, minv_ref, idx_ref)
        return carry

    lax.fori_loop(0, _GN // 2, _pipe, 0, unroll=2)

    part = jnp.sum(minv_ref[...])
    acc = jnp.where(m == 0, 0.0, loss_ref[0, 0]) + part
    loss_ref[0, 0] = jnp.where(m == _GM - 1, acc * _LOSS_SCALE, acc)


_dist_argmin = pl.pallas_call(
    _dist_argmin_body,
    grid=(_GM,),
    in_specs=[
        pl.BlockSpec((_BM, _D), lambda m: (m, 0)),
        pl.BlockSpec((_N, _D), lambda m: (0, 0)),
    ],
    out_specs=[
        pl.BlockSpec((1, 1, _BM), lambda m: (m, 0, 0)),
        pl.BlockSpec(memory_space=pltpu.SMEM),
    ],
    out_shape=[
        jax.ShapeDtypeStruct((_GM, 1, _BM), jnp.int32),
        jax.ShapeDtypeStruct((1, 1), jnp.float32),
    ],
    scratch_shapes=[
        pltpu.VMEM((1, _BM), jnp.float32),
        pltpu.VMEM((_N, 1), jnp.float32),
        pltpu.VMEM((_BN, _BM), jnp.float32),
        pltpu.VMEM((_BN, _BM), jnp.float32),
    ],
    compiler_params=pltpu.CompilerParams(
        dimension_semantics=("arbitrary",),
    ),
)


@functools.cache
def _sc_gather_fn():
    @functools.partial(
        pl.kernel,
        mesh=plsc.VectorSubcoreMesh(core_axis_name="c", subcore_axis_name="s"),
        out_type=jax.ShapeDtypeStruct((_B, _D), jnp.float32),
        scratch_types=[
            pltpu.VMEM((_NCH, _CH), jnp.int32),
            pltpu.VMEM((_RPW, _D), jnp.float32),
            pltpu.SemaphoreType.DMA,
        ],
    )
    def _sc_gather(w_hbm, idx_hbm, out_hbm, idx_v, rows_v, sem):
        wid = lax.axis_index("s") * 2 + lax.axis_index("c")
        pltpu.sync_copy(idx_hbm.at[pl.ds(wid * _NCH, _NCH)], idx_v)
        for k in range(_NCH):
            pltpu.async_copy(w_hbm.at[idx_v.at[k]],
                             rows_v.at[pl.ds(k * _CH, _CH)], sem).wait()
        pltpu.sync_copy(rows_v, out_hbm.at[pl.ds(wid * _RPW, _RPW)])

    return _sc_gather


def kernel(z, W):
    z_flat = z.reshape(-1, _D)
    idx, loss = _dist_argmin(z_flat, W)
    z_q = _sc_gather_fn()(W, idx.reshape(_B // _CH, _CH))
    return (z_q.reshape(z.shape), loss[0, 0])


# speedup vs baseline: 1.1794x; 1.1363x over previous
import functools

import jax
import jax.numpy as jnp
from jax import lax
from jax.experimental import pallas as pl
from jax.experimental.pallas import tpu as pltpu
from jax.experimental.pallas import tpu_sc as plsc

_D = 256
_N = 8192
_B = 8192
_BM = 1024
_BN = 1024
_GM = _B // _BM
_GN = _N // _BN
_LOSS_SCALE = 1.25 / (_B * _D)

_NW = 32
_RPW = _B // _NW
_CH = 128
_NCH = _RPW // _CH


def _merge(v1, i1, v2, i2):
    take2 = jnp.logical_or(v2 < v1, jnp.logical_and(v2 == v1, i2 < i1))
    return jnp.where(take2, v2, v1), jnp.where(take2, i2, i1)


def _consume(mm_ref, c, s_z_bc, w2_ref, minv_ref, idx_ref):
    sub_iota = lax.broadcasted_iota(jnp.int32, (8, _BM), 0)
    inf = jnp.full((8, _BM), jnp.inf, jnp.float32)
    zero = jnp.zeros((8, _BM), jnp.int32)
    accs = [[inf, zero], [inf, zero]]
    for j in range(_BN // 8):
        mm_j = mm_ref[pl.ds(8 * j, 8), :]
        w2_j = w2_ref[pl.ds(c * _BN + 8 * j, 8), :]
        v = (s_z_bc - mm_j) + jnp.broadcast_to(w2_j, (8, _BM))
        a = accs[j % 2]
        lt = v < a[0]
        a[0] = jnp.minimum(a[0], v)
        a[1] = jnp.where(lt, sub_iota + (8 * j), a[1])

    acc_val, acc_idx = _merge(accs[0][0], accs[0][1], accs[1][0], accs[1][1])
    for h in (4, 2, 1):
        acc_val, acc_idx = _merge(acc_val[:h], acc_idx[:h],
                                  acc_val[h:2 * h], acc_idx[h:2 * h])

    cmin = acc_val
    cidx = acc_idx + c * _BN

    better = jnp.logical_or(c == 0, cmin < minv_ref[...])
    new_min = jnp.where(better, cmin, minv_ref[...])
    minv_ref[...] = new_min
    idx_ref[...] = jnp.where(better[None], cidx[None], idx_ref[...])


def _dist_argmin_body(z_ref, w_ref, idx_ref, loss_ref,
                      minv_ref, w2_ref, mm0_ref, mm1_ref):
    m = pl.program_id(0)

    @pl.when(m == 0)
    def _():
        for c in range(_GN):
            w_blk0 = w_ref[pl.ds(c * _BN, _BN), :]
            w2_ref[pl.ds(c * _BN, _BN), :] = jnp.sum(
                w_blk0 * w_blk0, axis=1, keepdims=True)

    zsq = z_ref[...] * z_ref[...]
    ones_row = jnp.ones((1, _D), jnp.float32)
    s_z = lax.dot_general(ones_row, zsq, (((1,), (1,)), ((), ())),
                          preferred_element_type=jnp.float32)
    s_z_bc = jnp.broadcast_to(s_z, (8, _BM))

    z2_blk = z_ref[...] + z_ref[...]

    def _dot(c, out_ref):
        w_blk = w_ref[pl.ds(c * _BN, _BN), :]
        out_ref[...] = lax.dot_general(
            w_blk, z2_blk, (((1,), (1,)), ((), ())),
            preferred_element_type=jnp.float32)

    _dot(0, mm0_ref)

    def _pipe(k, carry):
        c0 = 2 * k
        _dot(c0 + 1, mm1_ref)
        _consume(mm0_ref, c0, s_z_bc, w2_ref, minv_ref, idx_ref)
        _dot(jnp.minimum(c0 + 2, _GN - 1), mm0_ref)
        _consume(mm1_ref, c0 + 1, s_z_bc, w2_ref, minv_ref, idx_ref)
        return carry

    for _k in range(_GN // 2):
        _pipe(_k, 0)

    part = jnp.sum(minv_ref[...])
    acc = jnp.where(m == 0, 0.0, loss_ref[0, 0]) + part
    loss_ref[0, 0] = jnp.where(m == _GM - 1, acc * _LOSS_SCALE, acc)


_dist_argmin = pl.pallas_call(
    _dist_argmin_body,
    grid=(_GM,),
    in_specs=[
        pl.BlockSpec((_BM, _D), lambda m: (m, 0)),
        pl.BlockSpec((_N, _D), lambda m: (0, 0)),
    ],
    out_specs=[
        pl.BlockSpec((1, 1, _BM), lambda m: (m, 0, 0)),
        pl.BlockSpec(memory_space=pltpu.SMEM),
    ],
    out_shape=[
        jax.ShapeDtypeStruct((_GM, 1, _BM), jnp.int32),
        jax.ShapeDtypeStruct((1, 1), jnp.float32),
    ],
    scratch_shapes=[
        pltpu.VMEM((1, _BM), jnp.float32),
        pltpu.VMEM((_N, 1), jnp.float32),
        pltpu.VMEM((_BN, _BM), jnp.float32),
        pltpu.VMEM((_BN, _BM), jnp.float32),
    ],
    compiler_params=pltpu.CompilerParams(
        dimension_semantics=("arbitrary",),
    ),
)


@functools.cache
def _sc_gather_fn():
    @functools.partial(
        pl.kernel,
        mesh=plsc.VectorSubcoreMesh(core_axis_name="c", subcore_axis_name="s"),
        out_type=jax.ShapeDtypeStruct((_B, _D), jnp.float32),
        scratch_types=[
            pltpu.VMEM((_NCH, _CH), jnp.int32),
            pltpu.VMEM((_RPW, _D), jnp.float32),
            pltpu.SemaphoreType.DMA,
        ],
    )
    def _sc_gather(w_hbm, idx_hbm, out_hbm, idx_v, rows_v, sem):
        wid = lax.axis_index("s") * 2 + lax.axis_index("c")
        pltpu.sync_copy(idx_hbm.at[pl.ds(wid * _NCH, _NCH)], idx_v)
        for k in range(_NCH):
            pltpu.async_copy(w_hbm.at[idx_v.at[k]],
                             rows_v.at[pl.ds(k * _CH, _CH)], sem).wait()
        pltpu.sync_copy(rows_v, out_hbm.at[pl.ds(wid * _RPW, _RPW)])

    return _sc_gather


def kernel(z, W):
    z_flat = z.reshape(-1, _D)
    idx, loss = _dist_argmin(z_flat, W)
    z_q = _sc_gather_fn()(W, idx.reshape(_B // _CH, _CH))
    return (z_q.reshape(z.shape), loss[0, 0])
